# baseline (speedup 1.0000x reference)
import jax
import jax.numpy as jnp
from jax.experimental import pallas as pl
from jax.experimental.pallas import tpu as pltpu
from jax.experimental.pallas import tpu_sc as plsc


_LANES = 16
_BH = 8
_NEWTON_ITERS = 1
_UNROLL = 2


def _newton_rsqrt(r2):
    i = jax.lax.bitcast_convert_type(r2, jnp.int32)
    i = jnp.int32(0x5F3759DF) - jax.lax.shift_right_logical(i, 1)
    y = jax.lax.bitcast_convert_type(i, jnp.float32)
    half = 0.5 * r2
    for _ in range(_NEWTON_ITERS):
        y = y * (1.5 - half * y * y)
    return y


def _compute_block(xr_vmem, xi_vmem, b_vmem, o0_vmem, o1_vmem):
    @plsc.parallel_loop(0, xr_vmem.shape[1], step=_LANES, unroll=_UNROLL)
    def _(c):
        b = b_vmem.at[pl.ds(0, 1), pl.ds(c, _LANES)][...]
        for r in range(_BH):
            slc = (pl.ds(r, 1), pl.ds(c, _LANES))
            xr = xr_vmem.at[slc][...]
            xi = xi_vmem.at[slc][...]
            r2 = xr * xr + xi * xi
            scale = jnp.maximum(1.0 + b * _newton_rsqrt(r2), 0.0)
            o0_vmem.at[slc][...] = scale * xr
            o1_vmem.at[slc][...] = scale * xi


def _sc_modrelu(x_real, x_imag, bias):
    n, c = x_real.shape
    mesh = plsc.VectorSubcoreMesh(core_axis_name="c", subcore_axis_name="s")
    n_tecs = 32
    rows_per_tec = n // n_tecs
    n_blocks = rows_per_tec // _BH
    f32 = x_real.dtype

    @pl.kernel(
        out_type=jax.ShapeDtypeStruct((2, n, c), f32),
        mesh=mesh,
        scratch_types=(
            [pltpu.VMEM((_BH, c), f32) for _ in range(4)]
            + [pltpu.VMEM((2, _BH, c), f32) for _ in range(2)]
            + [pltpu.VMEM((1, c), f32)]
            + [pltpu.SemaphoreType.DMA for _ in range(7)]
        ),
    )
    def run(xr_hbm, xi_hbm, b_hbm, o_hbm,
            xr0, xr1, xi0, xi1, ob0, ob1, bbuf,
            sir0, sir1, sii0, sii1, so0, so1, sb):
        tec = jax.lax.axis_index("c") * 16 + jax.lax.axis_index("s")
        base = tec * rows_per_tec

        in_bufs = ((xr0, xi0, sir0, sii0), (xr1, xi1, sir1, sii1))
        out_bufs = ((ob0, so0), (ob1, so1))

        def in_copies(i, p):
            rows = pl.ds(base + i * _BH, _BH)
            xr_b, xi_b, sr, si = in_bufs[p]
            cr = pltpu.make_async_copy(xr_hbm.at[rows], xr_b, sr)
            ci = pltpu.make_async_copy(xi_hbm.at[rows], xi_b, si)
            return cr, ci

        def out_copies(i, p):
            rows = pl.ds(base + i * _BH, _BH)
            o_b, s0 = out_bufs[p]
            c0 = pltpu.make_async_copy(o_b, o_hbm.at[:, rows, :], s0)
            return (c0,)

        cb = pltpu.make_async_copy(b_hbm, bbuf, sb)
        cb.start()
        cb.wait()
        for p in range(2):
            cr, ci = in_copies(p, p)
            cr.start()
            ci.start()

        @pl.loop(0, n_blocks, step=2)
        def _(i):
            for p in range(2):
                step = i + p
                cr, ci = in_copies(step, p)
                cr.wait()
                ci.wait()
                (co0,) = out_copies(step, p)

                @pl.when(step >= 2)
                def _():
                    (po0,) = out_copies(step - 2, p)
                    po0.wait()

                xr_b, xi_b, _, _ = in_bufs[p]
                o_b, _ = out_bufs[p]
                _compute_block(xr_b, xi_b, bbuf, o_b.at[0], o_b.at[1])
                co0.start()

                @pl.when(step + 2 < n_blocks)
                def _():
                    nr, ni = in_copies(step + 2, p)
                    nr.start()
                    ni.start()

        for p in range(2):
            (po0,) = out_copies(n_blocks - 2 + p, p)
            po0.wait()

    return run(x_real, x_imag, bias)


def kernel(x_real, x_imag, bias):
    return _sc_modrelu(x_real, x_imag, bias)


# speedup vs baseline: 1.0293x; 1.0293x over previous
import jax
import jax.numpy as jnp
from jax.experimental import pallas as pl
from jax.experimental.pallas import tpu as pltpu
from jax.experimental.pallas import tpu_sc as plsc


_LANES = 16
_BH = 8
_NEWTON_ITERS = 1
_UNROLL = 2


def _newton_rsqrt_over_sqrt2(r2):
    i = jax.lax.bitcast_convert_type(r2, jnp.int32)
    i = jnp.int32(0x5EF759DF) - jax.lax.shift_right_logical(i, 1)
    z = jax.lax.bitcast_convert_type(i, jnp.float32)
    for _ in range(_NEWTON_ITERS):
        z = z * (1.5 - r2 * z * z)
    return z


def _compute_block(xr_vmem, xi_vmem, b_vmem, o0_vmem, o1_vmem):
    @plsc.parallel_loop(0, xr_vmem.shape[1], step=_LANES, unroll=_UNROLL)
    def _(c):
        b = b_vmem.at[pl.ds(0, 1), pl.ds(c, _LANES)][...]
        b2 = b * jnp.float32(1.4142135623730951)
        for r in range(_BH):
            slc = (pl.ds(r, 1), pl.ds(c, _LANES))
            xr = xr_vmem.at[slc][...]
            xi = xi_vmem.at[slc][...]
            r2 = xr * xr + xi * xi
            scale = jnp.maximum(1.0 + b2 * _newton_rsqrt_over_sqrt2(r2), 0.0)
            o0_vmem.at[slc][...] = scale * xr
            o1_vmem.at[slc][...] = scale * xi


def _sc_modrelu(x_real, x_imag, bias):
    n, c = x_real.shape
    mesh = plsc.VectorSubcoreMesh(core_axis_name="c", subcore_axis_name="s")
    n_tecs = 32
    rows_per_tec = n // n_tecs
    n_blocks = rows_per_tec // _BH
    f32 = x_real.dtype

    @pl.kernel(
        out_type=jax.ShapeDtypeStruct((2, n, c), f32),
        mesh=mesh,
        scratch_types=(
            [pltpu.VMEM((_BH, c), f32) for _ in range(4)]
            + [pltpu.VMEM((2, _BH, c), f32) for _ in range(2)]
            + [pltpu.VMEM((1, c), f32)]
            + [pltpu.SemaphoreType.DMA for _ in range(7)]
        ),
    )
    def run(xr_hbm, xi_hbm, b_hbm, o_hbm,
            xr0, xr1, xi0, xi1, ob0, ob1, bbuf,
            sir0, sir1, sii0, sii1, so0, so1, sb):
        tec = jax.lax.axis_index("c") * 16 + jax.lax.axis_index("s")
        base = tec * rows_per_tec

        in_bufs = ((xr0, xi0, sir0, sii0), (xr1, xi1, sir1, sii1))
        out_bufs = ((ob0, so0), (ob1, so1))

        def in_copies(i, p):
            rows = pl.ds(base + i * _BH, _BH)
            xr_b, xi_b, sr, si = in_bufs[p]
            cr = pltpu.make_async_copy(xr_hbm.at[rows], xr_b, sr)
            ci = pltpu.make_async_copy(xi_hbm.at[rows], xi_b, si)
            return cr, ci

        def out_copies(i, p):
            rows = pl.ds(base + i * _BH, _BH)
            o_b, s0 = out_bufs[p]
            c0 = pltpu.make_async_copy(o_b, o_hbm.at[:, rows, :], s0)
            return (c0,)

        cb = pltpu.make_async_copy(b_hbm, bbuf, sb)
        cb.start()
        cb.wait()
        for p in range(2):
            cr, ci = in_copies(p, p)
            cr.start()
            ci.start()

        @pl.loop(0, n_blocks, step=2)
        def _(i):
            for p in range(2):
                step = i + p
                cr, ci = in_copies(step, p)
                cr.wait()
                ci.wait()
                (co0,) = out_copies(step, p)

                @pl.when(step >= 2)
                def _():
                    (po0,) = out_copies(step - 2, p)
                    po0.wait()

                xr_b, xi_b, _, _ = in_bufs[p]
                o_b, _ = out_bufs[p]
                _compute_block(xr_b, xi_b, bbuf, o_b.at[0], o_b.at[1])
                co0.start()

                @pl.when(step + 2 < n_blocks)
                def _():
                    nr, ni = in_copies(step + 2, p)
                    nr.start()
                    ni.start()

        for p in range(2):
            (po0,) = out_copies(n_blocks - 2 + p, p)
            po0.wait()

    return run(x_real, x_imag, bias)


def kernel(x_real, x_imag, bias):
    return _sc_modrelu(x_real, x_imag, bias)
